# baseline (speedup 1.0000x reference)
import jax
import jax.numpy as jnp
from jax import lax
from jax.experimental import pallas as pl
from jax.experimental.pallas import tpu as pltpu
from jax.experimental.pallas import tpu_sc as plsc

N = 10000
D = 128
E = 320000
NC, NS = 2, 16
NW = NC * NS
EPW = 20480
EP = NS * EPW
B = 128
GROUP = 16
ROWS_PER_TILE = 632
NPAD = ROWS_PER_TILE * NS
CW = 8
L = 16
NBANK = 2


def _mlp_block(x_ref, w_ref, b_ref, o_ref):
  o_ref[...] = jnp.maximum(
      jnp.dot(x_ref[...], w_ref[...], preferred_element_type=jnp.float32)
      + b_ref[...], 0.0)


def _node_messages(src_emb, w1t, b1row):
  grid = 10
  return pl.pallas_call(
      _mlp_block,
      grid=(grid,),
      in_specs=[
          pl.BlockSpec((N // grid, D), lambda i: (i, 0)),
          pl.BlockSpec((D, D), lambda i: (0, 0)),
          pl.BlockSpec((1, D), lambda i: (0, 0)),
      ],
      out_specs=pl.BlockSpec((N // grid, D), lambda i: (i, 0)),
      out_shape=jax.ShapeDtypeStruct((N, D), jnp.float32),
  )(src_emb, w1t, b1row)


def _sc_body(msg_hbm, src_hbm, dst_hbm,
             agg_out, cnt_out,
             src_v, dst_v, rows_v, ones_v, z16_v, agg_s, cnt_s,
             gsem, ssem, csem):
  c = lax.axis_index("c")
  s = lax.axis_index("s")
  base = s * ROWS_PER_TILE
  row0 = s * (EPW // B)
  ngroup = jnp.where(c == 0, EPW // (GROUP * B), 0)

  def fill_ones(i, carry):
    ones_v[pl.ds(i * 2, 2)] = jnp.full((2, CW), 1.0, jnp.float32)
    return carry
  lax.fori_loop(0, B // 2, fill_ones, 0)

  def fill_z16(i, carry):
    z16_v[pl.ds(i * 2, 2)] = jnp.zeros((2, CW), jnp.float32)
    return carry
  lax.fori_loop(0, ROWS_PER_TILE // 2, fill_z16, 0)

  def fill_zrows(i, carry):
    for j in range(D // L):
      rows_v[i, pl.ds(j * L, L)] = jnp.zeros((L,), jnp.float32)
    return carry
  lax.fori_loop(0, NBANK * B, fill_zrows, 0)

  nb = NBANK * B
  pltpu.sync_copy(rows_v, agg_s.at[pl.ds(base, nb)])
  pltpu.sync_copy(rows_v, agg_s.at[pl.ds(base + nb, nb)])
  tail = ROWS_PER_TILE - 2 * nb
  pltpu.sync_copy(rows_v.at[pl.ds(0, tail)],
                  agg_s.at[pl.ds(base + 2 * nb, tail)])
  pltpu.sync_copy(z16_v, cnt_s.at[pl.ds(base, ROWS_PER_TILE)])
  plsc.subcore_barrier()

  def bank(i):
    return rows_v.at[pl.ds((i % NBANK) * B, B)]

  def group(g, carry):
    r0 = row0 + g * GROUP
    pltpu.sync_copy(src_hbm.at[pl.ds(r0, GROUP)], src_v)
    pltpu.sync_copy(dst_hbm.at[pl.ds(r0, GROUP)], dst_v)

    gd = [None] * GROUP
    sd = [None] * GROUP
    cd = [None] * GROUP
    gd[0] = pltpu.async_copy(msg_hbm.at[src_v.at[0]], bank(0), gsem)
    for j in range(GROUP):
      if j >= 1:
        sd[j - 1].wait()
      if j + 1 < GROUP:
        gd[j + 1] = pltpu.async_copy(msg_hbm.at[src_v.at[j + 1]],
                                     bank(j + 1), gsem)
      gd[j].wait()
      sd[j] = pltpu.async_copy(bank(j), agg_s.at[dst_v.at[j]], ssem,
                               add=True)
      cd[j] = pltpu.async_copy(ones_v, cnt_s.at[dst_v.at[j]], csem,
                               add=True)
      if j >= 1:
        cd[j - 1].wait()
    sd[GROUP - 1].wait()
    cd[GROUP - 1].wait()
    return carry

  lax.fori_loop(0, ngroup, group, 0)
  plsc.subcore_barrier()

  pltpu.sync_copy(agg_s.at[pl.ds(base, nb)], rows_v)
  pltpu.sync_copy(rows_v, agg_out.at[c, pl.ds(base, nb)])
  pltpu.sync_copy(agg_s.at[pl.ds(base + nb, nb)], rows_v)
  pltpu.sync_copy(rows_v, agg_out.at[c, pl.ds(base + nb, nb)])
  pltpu.sync_copy(agg_s.at[pl.ds(base + 2 * nb, tail)],
                  rows_v.at[pl.ds(0, tail)])
  pltpu.sync_copy(rows_v.at[pl.ds(0, tail)],
                  agg_out.at[c, pl.ds(base + 2 * nb, tail)])
  pltpu.sync_copy(cnt_s.at[pl.ds(base, ROWS_PER_TILE)], z16_v)
  pltpu.sync_copy(z16_v, cnt_out.at[c, pl.ds(base, ROWS_PER_TILE)])


def _sc_aggregate(msg, srcp, dstp):
  mesh = plsc.VectorSubcoreMesh(core_axis_name="c", subcore_axis_name="s",
                                num_cores=NC, num_subcores=NS)
  f = pl.kernel(
      _sc_body,
      out_type=[
          jax.ShapeDtypeStruct((NC, NPAD, D), jnp.float32),
          jax.ShapeDtypeStruct((NC, NPAD, CW), jnp.float32),
      ],
      mesh=mesh,
      scratch_types=[
          pltpu.VMEM((GROUP, B), jnp.int32),
          pltpu.VMEM((GROUP, B), jnp.int32),
          pltpu.VMEM((NBANK * B, D), jnp.float32),
          pltpu.VMEM((B, CW), jnp.float32),
          pltpu.VMEM((ROWS_PER_TILE, CW), jnp.float32),
          pltpu.VMEM_SHARED((NPAD, D), jnp.float32),
          pltpu.VMEM_SHARED((NPAD, CW), jnp.float32),
          pltpu.SemaphoreType.DMA,
          pltpu.SemaphoreType.DMA,
          pltpu.SemaphoreType.DMA,
      ],
      compiler_params=pltpu.CompilerParams(use_tc_tiling_on_sc=False),
  )
  return f(msg, srcp, dstp)


def _update_block(dst_ref, agg_ref, cnt_ref, wa_ref, wb_ref, b_ref, o_ref):
  cnt = cnt_ref[0, :, :1] + cnt_ref[1, :, :1]
  agg = (agg_ref[0] + agg_ref[1]) / jnp.maximum(cnt, 1.0)
  acc = jnp.dot(dst_ref[...], wa_ref[...], preferred_element_type=jnp.float32)
  acc += jnp.dot(agg, wb_ref[...], preferred_element_type=jnp.float32)
  o_ref[...] = jnp.maximum(acc + b_ref[...], 0.0)


def _update(dst_emb, agg_parts, cnt_parts, w2at, w2bt, b2row):
  grid = 10
  r = N // grid
  return pl.pallas_call(
      _update_block,
      grid=(grid,),
      in_specs=[
          pl.BlockSpec((r, D), lambda i: (i, 0)),
          pl.BlockSpec((NC, r, D), lambda i: (0, i, 0)),
          pl.BlockSpec((NC, r, CW), lambda i: (0, i, 0)),
          pl.BlockSpec((D, D), lambda i: (0, 0)),
          pl.BlockSpec((D, D), lambda i: (0, 0)),
          pl.BlockSpec((1, D), lambda i: (0, 0)),
      ],
      out_specs=pl.BlockSpec((r, D), lambda i: (i, 0)),
      out_shape=jax.ShapeDtypeStruct((N, D), jnp.float32),
  )(dst_emb, agg_parts, cnt_parts, w2at, w2bt, b2row)


def kernel(src_embeddings, dst_embeddings, edge_index, W1, b1, W2, b2):
  src_idx = edge_index[0]
  dst_idx = edge_index[1]
  pad = EP - E
  srcp = jnp.pad(src_idx, (0, pad)).reshape(EP // B, B)
  pad_dst = N + jnp.arange(pad, dtype=jnp.int32) % (NPAD - N)
  dstp = jnp.concatenate([dst_idx, pad_dst]).reshape(EP // B, B)

  msg = _node_messages(src_embeddings, W1.T, b1.reshape(1, D))
  agg_parts, cnt_parts = _sc_aggregate(msg, srcp, dstp)
  return _update(dst_embeddings, agg_parts, cnt_parts,
                 W2[:, :D].T, W2[:, D:].T, b2.reshape(1, D))


# speedup vs baseline: 2.2291x; 2.2291x over previous
import jax
import jax.numpy as jnp
from jax import lax
from jax.experimental import pallas as pl
from jax.experimental.pallas import tpu as pltpu
from jax.experimental.pallas import tpu_sc as plsc

N = 10000
D = 128
E = 320000
NC, NS = 2, 16
NW = NC * NS
EPW = 20480
EP = NS * EPW
B = 128
GROUP = 16
ROWS_PER_TILE = 632
NPAD = ROWS_PER_TILE * NS
CW = 8
L = 16
NBANK = 2


def _mlp_block(x_ref, w_ref, b_ref, o_ref):
  o_ref[...] = jnp.maximum(
      jnp.dot(x_ref[...], w_ref[...], preferred_element_type=jnp.float32)
      + b_ref[...], 0.0)


def _node_messages(src_emb, w1t, b1row):
  grid = 10
  return pl.pallas_call(
      _mlp_block,
      grid=(grid,),
      in_specs=[
          pl.BlockSpec((N // grid, D), lambda i: (i, 0)),
          pl.BlockSpec((D, D), lambda i: (0, 0)),
          pl.BlockSpec((1, D), lambda i: (0, 0)),
      ],
      out_specs=pl.BlockSpec((N // grid, D), lambda i: (i, 0)),
      out_shape=jax.ShapeDtypeStruct((N, D), jnp.float32),
  )(src_emb, w1t, b1row)


def _sc_body(msg_hbm, src_hbm, dst_hbm,
             agg_out, cnt_out,
             src_v, dst_v, rows_v, ones_v, z16_v, agg_s, cnt_s,
             gsem, ssem, csem):
  c = lax.axis_index("c")
  s = lax.axis_index("s")
  base = s * ROWS_PER_TILE
  row0 = s * (EPW // B)
  ngroup = jnp.where(c == 0, EPW // (GROUP * B), 0)

  def fill_ones(i, carry):
    ones_v[pl.ds(i * 2, 2)] = jnp.full((2, CW), 1.0, jnp.float32)
    return carry
  lax.fori_loop(0, B // 2, fill_ones, 0)

  def fill_z16(i, carry):
    z16_v[pl.ds(i * 2, 2)] = jnp.zeros((2, CW), jnp.float32)
    return carry
  lax.fori_loop(0, ROWS_PER_TILE // 2, fill_z16, 0)

  def fill_zrows(i, carry):
    for j in range(D // L):
      rows_v[i, pl.ds(j * L, L)] = jnp.zeros((L,), jnp.float32)
    return carry
  lax.fori_loop(0, NBANK * B, fill_zrows, 0)

  nb = NBANK * B
  pltpu.sync_copy(rows_v, agg_s.at[pl.ds(base, nb)])
  pltpu.sync_copy(rows_v, agg_s.at[pl.ds(base + nb, nb)])
  tail = ROWS_PER_TILE - 2 * nb
  pltpu.sync_copy(rows_v.at[pl.ds(0, tail)],
                  agg_s.at[pl.ds(base + 2 * nb, tail)])
  pltpu.sync_copy(z16_v, cnt_s.at[pl.ds(base, ROWS_PER_TILE)])
  plsc.subcore_barrier()

  def bank(i):
    return rows_v.at[pl.ds((i % NBANK) * B, B)]

  def group(g, carry):
    r0 = row0 + g * GROUP
    pltpu.sync_copy(src_hbm.at[pl.ds(r0, GROUP)], src_v)
    pltpu.sync_copy(dst_hbm.at[pl.ds(r0, GROUP)], dst_v)

    gd = [None] * GROUP
    sd = [None] * GROUP
    cd = [None] * GROUP
    gd[0] = pltpu.async_copy(msg_hbm.at[src_v.at[0]], bank(0), gsem)
    for j in range(GROUP):
      if j >= 1:
        sd[j - 1].wait()
      if j + 1 < GROUP:
        gd[j + 1] = pltpu.async_copy(msg_hbm.at[src_v.at[j + 1]],
                                     bank(j + 1), gsem)
      gd[j].wait()
      sd[j] = pltpu.async_copy(bank(j), agg_s.at[dst_v.at[j]], ssem,
                               add=True)
      cd[j] = pltpu.async_copy(ones_v, cnt_s.at[dst_v.at[j]], csem,
                               add=True)
      if j >= 1:
        cd[j - 1].wait()
    sd[GROUP - 1].wait()
    cd[GROUP - 1].wait()
    return carry

  lax.fori_loop(0, ngroup, group, 0)
  plsc.subcore_barrier()

  pltpu.sync_copy(agg_s.at[pl.ds(base, nb)], rows_v)
  pltpu.sync_copy(rows_v, agg_out.at[c, pl.ds(base, nb)])
  pltpu.sync_copy(agg_s.at[pl.ds(base + nb, nb)], rows_v)
  pltpu.sync_copy(rows_v, agg_out.at[c, pl.ds(base + nb, nb)])
  pltpu.sync_copy(agg_s.at[pl.ds(base + 2 * nb, tail)],
                  rows_v.at[pl.ds(0, tail)])
  pltpu.sync_copy(rows_v.at[pl.ds(0, tail)],
                  agg_out.at[c, pl.ds(base + 2 * nb, tail)])
  pltpu.sync_copy(cnt_s.at[pl.ds(base, ROWS_PER_TILE)], z16_v)
  pltpu.sync_copy(z16_v, cnt_out.at[c, pl.ds(base, ROWS_PER_TILE)])


def _sc_aggregate(msg, srcp, dstp):
  mesh = plsc.VectorSubcoreMesh(core_axis_name="c", subcore_axis_name="s",
                                num_cores=NC, num_subcores=NS)
  f = pl.kernel(
      _sc_body,
      out_type=[
          jax.ShapeDtypeStruct((NC, NPAD, D), jnp.float32),
          jax.ShapeDtypeStruct((NC, NPAD, CW), jnp.float32),
      ],
      mesh=mesh,
      scratch_types=[
          pltpu.VMEM((GROUP, B), jnp.int32),
          pltpu.VMEM((GROUP, B), jnp.int32),
          pltpu.VMEM((NBANK * B, D), jnp.float32),
          pltpu.VMEM((B, CW), jnp.float32),
          pltpu.VMEM((ROWS_PER_TILE, CW), jnp.float32),
          pltpu.VMEM_SHARED((NPAD, D), jnp.float32),
          pltpu.VMEM_SHARED((NPAD, CW), jnp.float32),
          pltpu.SemaphoreType.DMA,
          pltpu.SemaphoreType.DMA,
          pltpu.SemaphoreType.DMA,
      ],
      compiler_params=pltpu.CompilerParams(use_tc_tiling_on_sc=False),
  )
  return f(msg, srcp, dstp)


def _update_block(dst_ref, agg_ref, cnt_ref, wa_ref, wb_ref, b_ref, o_ref):
  cnt = cnt_ref[0, :, :1] + cnt_ref[1, :, :1]
  agg = (agg_ref[0] + agg_ref[1]) / jnp.maximum(cnt, 1.0)
  acc = jnp.dot(dst_ref[...], wa_ref[...], preferred_element_type=jnp.float32)
  acc += jnp.dot(agg, wb_ref[...], preferred_element_type=jnp.float32)
  o_ref[...] = jnp.maximum(acc + b_ref[...], 0.0)


def _update(dst_emb, agg_parts, cnt_parts, w2at, w2bt, b2row):
  grid = 10
  r = N // grid
  return pl.pallas_call(
      _update_block,
      grid=(grid,),
      in_specs=[
          pl.BlockSpec((r, D), lambda i: (i, 0)),
          pl.BlockSpec((NC, r, D), lambda i: (0, i, 0)),
          pl.BlockSpec((NC, r, CW), lambda i: (0, i, 0)),
          pl.BlockSpec((D, D), lambda i: (0, 0)),
          pl.BlockSpec((D, D), lambda i: (0, 0)),
          pl.BlockSpec((1, D), lambda i: (0, 0)),
      ],
      out_specs=pl.BlockSpec((r, D), lambda i: (i, 0)),
      out_shape=jax.ShapeDtypeStruct((N, D), jnp.float32),
  )(dst_emb, agg_parts, cnt_parts, w2at, w2bt, b2row)


def kernel(src_embeddings, dst_embeddings, edge_index, W1, b1, W2, b2):
  src_idx = edge_index[0]
  dst_idx = edge_index[1]
  pad = EP - E
  pad_src = jnp.arange(pad, dtype=jnp.int32) % N
  srcp = jnp.concatenate([src_idx, pad_src]).reshape(EP // B, B)
  pad_dst = N + jnp.arange(pad, dtype=jnp.int32) % (NPAD - N)
  dstp = jnp.concatenate([dst_idx, pad_dst]).reshape(EP // B, B)

  msg = _node_messages(src_embeddings, W1.T, b1.reshape(1, D))
  agg_parts, cnt_parts = _sc_aggregate(msg, srcp, dstp)
  return _update(dst_embeddings, agg_parts, cnt_parts,
                 W2[:, :D].T, W2[:, D:].T, b2.reshape(1, D))


# speedup vs baseline: 3.4388x; 1.5427x over previous
import jax
import jax.numpy as jnp
from jax import lax
from jax.experimental import pallas as pl
from jax.experimental.pallas import tpu as pltpu
from jax.experimental.pallas import tpu_sc as plsc

N = 10000
D = 128
E = 320000
NC, NS = 2, 16
NW = NC * NS
EPW = 10240
EP = NW * EPW
B = 128
GROUP = 16
ROWS_PER_TILE = 632
NPAD = ROWS_PER_TILE * NS
CW = 8
L = 16
NBANK = 2


def _mlp_block(x_ref, w_ref, b_ref, o_ref):
  o_ref[...] = jnp.maximum(
      jnp.dot(x_ref[...], w_ref[...], preferred_element_type=jnp.float32)
      + b_ref[...], 0.0)


def _node_messages(src_emb, w1t, b1row):
  grid = 10
  return pl.pallas_call(
      _mlp_block,
      grid=(grid,),
      in_specs=[
          pl.BlockSpec((N // grid, D), lambda i: (i, 0)),
          pl.BlockSpec((D, D), lambda i: (0, 0)),
          pl.BlockSpec((1, D), lambda i: (0, 0)),
      ],
      out_specs=pl.BlockSpec((N // grid, D), lambda i: (i, 0)),
      out_shape=jax.ShapeDtypeStruct((N, D), jnp.float32),
  )(src_emb, w1t, b1row)


def _sc_body(msg_hbm, src_hbm, dst_hbm,
             agg_out, cnt_out,
             src_v, dst_v, rows_v, ones_v, z16_v, agg_s, cnt_s,
             gsem, ssem, csem):
  c = lax.axis_index("c")
  s = lax.axis_index("s")
  base = s * ROWS_PER_TILE
  row0 = (c * NS + s) * (EPW // B)
  ngroup = EPW // (GROUP * B)

  def fill_ones(i, carry):
    ones_v[pl.ds(i * 2, 2)] = jnp.full((2, CW), 1.0, jnp.float32)
    return carry
  lax.fori_loop(0, B // 2, fill_ones, 0)

  def fill_z16(i, carry):
    z16_v[pl.ds(i * 2, 2)] = jnp.zeros((2, CW), jnp.float32)
    return carry
  lax.fori_loop(0, ROWS_PER_TILE // 2, fill_z16, 0)

  def fill_zrows(i, carry):
    for j in range(D // L):
      rows_v[i, pl.ds(j * L, L)] = jnp.zeros((L,), jnp.float32)
    return carry
  lax.fori_loop(0, NBANK * B, fill_zrows, 0)

  nb = NBANK * B
  pltpu.sync_copy(rows_v, agg_s.at[pl.ds(base, nb)])
  pltpu.sync_copy(rows_v, agg_s.at[pl.ds(base + nb, nb)])
  tail = ROWS_PER_TILE - 2 * nb
  pltpu.sync_copy(rows_v.at[pl.ds(0, tail)],
                  agg_s.at[pl.ds(base + 2 * nb, tail)])
  pltpu.sync_copy(z16_v, cnt_s.at[pl.ds(base, ROWS_PER_TILE)])
  plsc.subcore_barrier()

  def bank(i):
    return rows_v.at[pl.ds((i % NBANK) * B, B)]

  def group(g, carry):
    r0 = row0 + g * GROUP
    pltpu.sync_copy(src_hbm.at[pl.ds(r0, GROUP)], src_v)
    pltpu.sync_copy(dst_hbm.at[pl.ds(r0, GROUP)], dst_v)

    gd = [None] * GROUP
    sd = [None] * GROUP
    cd = [None] * GROUP
    gd[0] = pltpu.async_copy(msg_hbm.at[src_v.at[0]], bank(0), gsem)
    for j in range(GROUP):
      if j >= 1:
        sd[j - 1].wait()
      if j + 1 < GROUP:
        gd[j + 1] = pltpu.async_copy(msg_hbm.at[src_v.at[j + 1]],
                                     bank(j + 1), gsem)
      gd[j].wait()
      sd[j] = pltpu.async_copy(bank(j), agg_s.at[dst_v.at[j]], ssem,
                               add=True)
      cd[j] = pltpu.async_copy(ones_v, cnt_s.at[dst_v.at[j]], csem,
                               add=True)
      if j >= 1:
        cd[j - 1].wait()
    sd[GROUP - 1].wait()
    cd[GROUP - 1].wait()
    return carry

  lax.fori_loop(0, ngroup, group, 0)
  plsc.subcore_barrier()

  pltpu.sync_copy(agg_s.at[pl.ds(base, nb)], rows_v)
  pltpu.sync_copy(rows_v, agg_out.at[c, pl.ds(base, nb)])
  pltpu.sync_copy(agg_s.at[pl.ds(base + nb, nb)], rows_v)
  pltpu.sync_copy(rows_v, agg_out.at[c, pl.ds(base + nb, nb)])
  pltpu.sync_copy(agg_s.at[pl.ds(base + 2 * nb, tail)],
                  rows_v.at[pl.ds(0, tail)])
  pltpu.sync_copy(rows_v.at[pl.ds(0, tail)],
                  agg_out.at[c, pl.ds(base + 2 * nb, tail)])
  pltpu.sync_copy(cnt_s.at[pl.ds(base, ROWS_PER_TILE)], z16_v)
  pltpu.sync_copy(z16_v, cnt_out.at[c, pl.ds(base, ROWS_PER_TILE)])


def _sc_aggregate(msg, srcp, dstp):
  mesh = plsc.VectorSubcoreMesh(core_axis_name="c", subcore_axis_name="s",
                                num_cores=NC, num_subcores=NS)
  f = pl.kernel(
      _sc_body,
      out_type=[
          jax.ShapeDtypeStruct((NC, NPAD, D), jnp.float32),
          jax.ShapeDtypeStruct((NC, NPAD, CW), jnp.float32),
      ],
      mesh=mesh,
      scratch_types=[
          pltpu.VMEM((GROUP, B), jnp.int32),
          pltpu.VMEM((GROUP, B), jnp.int32),
          pltpu.VMEM((NBANK * B, D), jnp.float32),
          pltpu.VMEM((B, CW), jnp.float32),
          pltpu.VMEM((ROWS_PER_TILE, CW), jnp.float32),
          pltpu.VMEM_SHARED((NPAD, D), jnp.float32),
          pltpu.VMEM_SHARED((NPAD, CW), jnp.float32),
          pltpu.SemaphoreType.DMA,
          pltpu.SemaphoreType.DMA,
          pltpu.SemaphoreType.DMA,
      ],
      compiler_params=pltpu.CompilerParams(use_tc_tiling_on_sc=False),
  )
  return f(msg, srcp, dstp)


def _update_block(dst_ref, agg_ref, cnt_ref, wa_ref, wb_ref, b_ref, o_ref):
  cnt = cnt_ref[0, :, :1] + cnt_ref[1, :, :1]
  agg = (agg_ref[0] + agg_ref[1]) / jnp.maximum(cnt, 1.0)
  acc = jnp.dot(dst_ref[...], wa_ref[...], preferred_element_type=jnp.float32)
  acc += jnp.dot(agg, wb_ref[...], preferred_element_type=jnp.float32)
  o_ref[...] = jnp.maximum(acc + b_ref[...], 0.0)


def _update(dst_emb, agg_parts, cnt_parts, w2at, w2bt, b2row):
  grid = 10
  r = N // grid
  return pl.pallas_call(
      _update_block,
      grid=(grid,),
      in_specs=[
          pl.BlockSpec((r, D), lambda i: (i, 0)),
          pl.BlockSpec((NC, r, D), lambda i: (0, i, 0)),
          pl.BlockSpec((NC, r, CW), lambda i: (0, i, 0)),
          pl.BlockSpec((D, D), lambda i: (0, 0)),
          pl.BlockSpec((D, D), lambda i: (0, 0)),
          pl.BlockSpec((1, D), lambda i: (0, 0)),
      ],
      out_specs=pl.BlockSpec((r, D), lambda i: (i, 0)),
      out_shape=jax.ShapeDtypeStruct((N, D), jnp.float32),
  )(dst_emb, agg_parts, cnt_parts, w2at, w2bt, b2row)


def kernel(src_embeddings, dst_embeddings, edge_index, W1, b1, W2, b2):
  src_idx = edge_index[0]
  dst_idx = edge_index[1]
  pad = EP - E
  pad_src = jnp.arange(pad, dtype=jnp.int32) % N
  srcp = jnp.concatenate([src_idx, pad_src]).reshape(EP // B, B)
  pad_dst = N + jnp.arange(pad, dtype=jnp.int32) % (NPAD - N)
  dstp = jnp.concatenate([dst_idx, pad_dst]).reshape(EP // B, B)

  msg = _node_messages(src_embeddings, W1.T, b1.reshape(1, D))
  agg_parts, cnt_parts = _sc_aggregate(msg, srcp, dstp)
  return _update(dst_embeddings, agg_parts, cnt_parts,
                 W2[:, :D].T, W2[:, D:].T, b2.reshape(1, D))
